# baseline (speedup 1.0000x reference)
import functools

import jax
import jax.numpy as jnp
from jax import lax
from jax.experimental import pallas as pl
from jax.experimental.pallas import tpu as pltpu
from jax.experimental.pallas import tpu_sc as plsc

_info = plsc.get_sparse_core_info()
_NC, _NS = _info.num_cores, _info.num_subcores
_NW = _NC * _NS


def _make_gather(vocab: int, hdim: int, batch: int):
    assert batch % _NW == 0
    b_per_w = batch // _NW
    mesh = plsc.VectorSubcoreMesh(core_axis_name="c", subcore_axis_name="s")

    @functools.partial(
        pl.kernel,
        mesh=mesh,
        out_type=jax.ShapeDtypeStruct((batch, hdim), jnp.float32),
        scratch_types=[
            pltpu.VMEM((b_per_w,), jnp.int32),
            pltpu.VMEM((b_per_w, hdim), jnp.float32),
            pltpu.SemaphoreType.DMA,
        ],
        compiler_params=pltpu.CompilerParams(use_tc_tiling_on_sc=False),
    )
    def gather_kernel(table_hbm, idx_hbm, out_hbm, idx_v, rows_v, sem):
        wid = lax.axis_index("s") * _NC + lax.axis_index("c")
        base = wid * b_per_w
        pltpu.sync_copy(idx_hbm.at[pl.ds(base, b_per_w)], idx_v)
        pltpu.async_copy(table_hbm.at[idx_v], rows_v, sem).wait()
        pltpu.sync_copy(rows_v, out_hbm.at[pl.ds(base, b_per_w)])

    return gather_kernel


def kernel(ordered_embs, nodes):
    vocab, hdim = ordered_embs.shape
    batch = nodes.shape[0]
    idx = nodes.reshape(batch)
    return _make_gather(vocab, hdim, batch)(ordered_embs, idx)


# speedup vs baseline: 7.0689x; 7.0689x over previous
import functools

import jax
import jax.numpy as jnp
from jax import lax
from jax.experimental import pallas as pl
from jax.experimental.pallas import tpu as pltpu
from jax.experimental.pallas import tpu_sc as plsc

_info = plsc.get_sparse_core_info()
_NC, _NS, _NL = _info.num_cores, _info.num_subcores, _info.num_lanes
_NW = _NC * _NS

_CHUNK = 16


def _make_gather(vocab: int, hdim: int, batch: int):
    assert batch % (_NW * _CHUNK) == 0
    b_per_w = batch // _NW
    n_chunks = b_per_w // _CHUNK
    mesh = plsc.VectorSubcoreMesh(core_axis_name="c", subcore_axis_name="s")

    @functools.partial(
        pl.kernel,
        mesh=mesh,
        out_type=jax.ShapeDtypeStruct((hdim, batch), jnp.float32),
        scratch_types=[
            pltpu.VMEM((b_per_w,), jnp.int32),
            pltpu.VMEM((hdim, b_per_w), jnp.float32),
            pltpu.VMEM((2, _CHUNK, hdim, 128), jnp.float32),
            pltpu.SemaphoreType.DMA,
        ],
        compiler_params=pltpu.CompilerParams(needs_layout_passes=False),
    )
    def gather_kernel(table_hbm, idx_hbm, out_hbm, idx_v, cols_v, ring_v, sem):
        wid = lax.axis_index("s") * _NC + lax.axis_index("c")
        base = pl.multiple_of(wid * b_per_w, 128)
        pltpu.sync_copy(idx_hbm.at[pl.ds(base, b_per_w)], idx_v)

        row_iota = lax.iota(jnp.int32, hdim)

        def fire(c, slot):
            vs = idx_v[pl.ds(c * _CHUNK, _CHUNK)]
            for jj in range(_CHUNK):
                v = vs[jj]
                col0 = pl.multiple_of(
                    lax.shift_left(lax.shift_right_logical(v, 7), 7), 128
                )
                pltpu.async_copy(
                    table_hbm.at[:, pl.ds(col0, 128)],
                    ring_v.at[slot, jj],
                    sem,
                )

        def drain_extract(c, slot):
            vs = idx_v[pl.ds(c * _CHUNK, _CHUNK)]
            for jj in range(_CHUNK):
                pltpu.make_async_copy(
                    table_hbm.at[:, pl.ds(0, 128)],
                    ring_v.at[slot, jj],
                    sem,
                ).wait()
            for jj in range(_CHUNK):
                v = vs[jj]
                r = lax.bitwise_and(v, 127)
                col = plsc.load_gather(
                    ring_v.at[slot, jj],
                    [row_iota, jnp.full((hdim,), r, jnp.int32)],
                )
                plsc.store_scatter(
                    cols_v,
                    [row_iota, jnp.full((hdim,), c * _CHUNK + jj, jnp.int32)],
                    col,
                )

        fire(0, 0)

        def chunk_body(c, carry):
            slot = lax.rem(c, 2)

            @pl.when(c + 1 < n_chunks)
            def _():
                fire(c + 1, 1 - slot)

            drain_extract(c, slot)
            return carry

        lax.fori_loop(0, n_chunks, chunk_body, 0, unroll=False)
        pltpu.sync_copy(cols_v, out_hbm.at[:, pl.ds(base, b_per_w)])

    return gather_kernel


def kernel(ordered_embs, nodes):
    vocab, hdim = ordered_embs.shape
    batch = nodes.shape[0]
    idx = nodes.reshape(batch)
    out_t = _make_gather(vocab, hdim, batch)(ordered_embs.T, idx)
    return out_t.T
